# baseline (speedup 1.0000x reference)
import functools

import jax
import jax.numpy as jnp
from jax import lax
from jax.experimental import pallas as pl
from jax.experimental.pallas import tpu as pltpu
from jax.experimental.pallas import tpu_sc as plsc

_DIM = 64
_NC = 2
_NS = 16
_NW = _NC * _NS
_PANEL = 128
_NBUF = 5


@functools.partial(jax.jit, static_argnames=("batch", "hist"))
def _sc_gather(xt_flat, table, batch, hist):
    nbb = batch // _PANEL
    n_panels = hist * nbb
    n_per_w = n_panels // _NW
    assert n_per_w % _NBUF == 0
    mesh = plsc.VectorSubcoreMesh(core_axis_name="c", subcore_axis_name="s")

    @functools.partial(
        pl.kernel,
        mesh=mesh,
        out_type=jax.ShapeDtypeStruct((hist, _DIM // 8, nbb, 1024), jnp.float32),
        scratch_types=[
            pltpu.VMEM((n_per_w * _PANEL,), jnp.int32),
            pltpu.VMEM((_NBUF, _PANEL, _DIM), jnp.float32),
            pltpu.VMEM((_NBUF, 8192), jnp.float32),
            pltpu.SemaphoreType.DMA((_NBUF,)),
            pltpu.SemaphoreType.DMA((_NBUF,)),
        ],
        compiler_params=pltpu.CompilerParams(
            use_tc_tiling_on_sc=False, needs_layout_passes=False
        ),
    )
    def k(x_hbm, table_hbm, out_hbm, idx_v, rows_v, tr_v, gsem, osem):
        wid = lax.axis_index("s") * _NC + lax.axis_index("c")
        p0 = wid * n_per_w
        pltpu.sync_copy(x_hbm.at[pl.ds(p0 * _PANEL, n_per_w * _PANEL)], idx_v)

        iota = lax.iota(jnp.int32, 16)
        pre = [(c + iota) * _PANEL for c in (0, 16, 32, 48)]

        def fire_gather(m, slot):
            pltpu.async_copy(
                table_hbm.at[idx_v.at[pl.ds(m * _PANEL, _PANEL)]],
                rows_v.at[slot],
                gsem.at[slot],
            )

        def drain_gather(slot):
            pltpu.make_async_copy(
                table_hbm.at[pl.ds(0, _PANEL)],
                rows_v.at[slot],
                gsem.at[slot],
            ).wait()

        def transpose_panel(slot):
            rows = rows_v.at[slot]
            tr = tr_v.at[slot]

            @pl.loop(0, _PANEL, unroll=8)
            def _row(l):
                dst = [p + l for p in pre]
                for c in range(4):
                    v = rows[l, pl.ds(c * 16, 16)]
                    plsc.store_scatter(tr, [dst[c]], v)

        def fire_out(m, slot):
            p = p0 + m
            t = p // nbb
            bb = lax.rem(p, nbb)
            for fb in range(8):
                pltpu.async_copy(
                    tr_v.at[slot, pl.ds(fb * 1024, 1024)],
                    out_hbm.at[t, fb, bb],
                    osem.at[slot],
                )

        def drain_out(slot):
            for fb in range(8):
                pltpu.make_async_copy(
                    tr_v.at[slot, pl.ds(fb * 1024, 1024)],
                    out_hbm.at[0, fb, 0],
                    osem.at[slot],
                ).wait()

        for b in range(_NBUF - 1):
            fire_gather(b, b)

        @pl.loop(0, n_per_w, step=_NBUF)
        def _outer(g):
            for b in range(_NBUF):
                m = g + b
                sf = (b - 1) % _NBUF

                @pl.when(m >= 1)
                def _():
                    drain_out(sf)

                @pl.when(m + _NBUF - 1 < n_per_w)
                def _():
                    fire_gather(m + _NBUF - 1, sf)

                drain_gather(b)
                transpose_panel(b)
                fire_out(m, b)

        drain_out(_NBUF - 1)

    return k(xt_flat, table)


def kernel(x, table):
    batch, hist = x.shape
    xt = x.T.reshape(-1).astype(jnp.int32)
    t5 = _sc_gather(xt, table, batch, hist)
    out = (
        t5.reshape(hist, _DIM // 8, batch // _PANEL, 8, _PANEL)
        .transpose(2, 4, 0, 1, 3)
        .reshape(batch, hist, _DIM)
    )
    return out


# speedup vs baseline: 1.1307x; 1.1307x over previous
import functools

import jax
import jax.numpy as jnp
from jax import lax
from jax.experimental import pallas as pl
from jax.experimental.pallas import tpu as pltpu
from jax.experimental.pallas import tpu_sc as plsc

_DIM = 64
_NC = 2
_NS = 16
_NW = _NC * _NS
_CHUNK = 256
_NBUF = 5


@functools.partial(jax.jit, static_argnames=("total",))
def _sc_gather(x_flat, table, total):
    n_per_w = total // _NW
    n_chunks = n_per_w // _CHUNK
    assert n_chunks % _NBUF == 0
    mesh = plsc.VectorSubcoreMesh(core_axis_name="c", subcore_axis_name="s")

    @functools.partial(
        pl.kernel,
        mesh=mesh,
        out_type=jax.ShapeDtypeStruct((total, _DIM), jnp.float32),
        scratch_types=[
            pltpu.VMEM((n_per_w,), jnp.int32),
            pltpu.VMEM((_NBUF, _CHUNK, _DIM), jnp.float32),
            pltpu.SemaphoreType.DMA((_NBUF,)),
            pltpu.SemaphoreType.DMA((_NBUF,)),
        ],
        compiler_params=pltpu.CompilerParams(use_tc_tiling_on_sc=False),
    )
    def k(x_hbm, table_hbm, out_hbm, idx_v, rows_v, gsem, osem):
        wid = lax.axis_index("s") * _NC + lax.axis_index("c")
        base = wid * n_per_w
        pltpu.sync_copy(x_hbm.at[pl.ds(base, n_per_w)], idx_v)

        for b in range(_NBUF - 1):
            pltpu.async_copy(
                table_hbm.at[idx_v.at[pl.ds(b * _CHUNK, _CHUNK)]],
                rows_v.at[b],
                gsem.at[b],
            )

        @pl.loop(0, n_chunks, step=_NBUF)
        def _outer(g):
            for b in range(_NBUF):
                j = g + b
                sf = (b - 1) % _NBUF

                @pl.when(j >= 1)
                def _():
                    pltpu.make_async_copy(
                        rows_v.at[sf],
                        out_hbm.at[pl.ds(base, _CHUNK)],
                        osem.at[sf],
                    ).wait()

                @pl.when(j + _NBUF - 1 < n_chunks)
                def _():
                    pltpu.async_copy(
                        table_hbm.at[idx_v.at[pl.ds((j + _NBUF - 1) * _CHUNK, _CHUNK)]],
                        rows_v.at[sf],
                        gsem.at[sf],
                    )

                pltpu.make_async_copy(
                    table_hbm.at[idx_v.at[pl.ds(j * _CHUNK, _CHUNK)]],
                    rows_v.at[b],
                    gsem.at[b],
                ).wait()
                pltpu.async_copy(
                    rows_v.at[b],
                    out_hbm.at[pl.ds(base + j * _CHUNK, _CHUNK)],
                    osem.at[b],
                )

        pltpu.make_async_copy(
            rows_v.at[_NBUF - 1],
            out_hbm.at[pl.ds(base, _CHUNK)],
            osem.at[_NBUF - 1],
        ).wait()

    return k(x_flat, table)


def kernel(x, table):
    batch, hist = x.shape
    total = batch * hist
    xf = x.reshape(-1).astype(jnp.int32)
    out = _sc_gather(xf, table, total)
    return out.reshape(batch, hist, _DIM)
